# Initial kernel scaffold; baseline (speedup 1.0000x reference)
import jax
import jax.numpy as jnp
from jax.experimental import pallas as pl


def kernel(node_features, edge_features, edge_index, node_tiers, wn_score, we_score, Wq, Wk, Wv, Wo, Wnk, Wc1, bc1, Wc2, bc2):
    raise NotImplementedError("write your pallas kernel here")


# speedup vs baseline: 4.0556x; 4.0556x over previous
import jax
import jax.numpy as jnp
import numpy as np
from jax.experimental import pallas as pl

_N = 10000
_E = 320000
_DN = 128
_DE = 16
_H = 4
_DH = 4
_C = 8
_KN = 4000
_KE = 128000
_BN = 1000
_BE = 8000
_GN = _N // _BN
_GE = _E // _BE
_NPAD = 80 * 128 - _N
_INT_MIN = np.int32(-2147483648)
_INT_MAX = np.int32(2147483647)

_S_np = np.zeros((_DE, _H), np.float32)
for _h in range(_H):
    _S_np[_h * _DH:(_h + 1) * _DH, _h] = 1.0
_R_np = _S_np.T.copy()


def _node_kernel(x_ref, wn_ref, wnk_ref, ns_ref, xn_ref):
    x = x_ref[...]
    ns_ref[...] = jnp.dot(x, wn_ref[...], preferred_element_type=jnp.float32)
    xn_ref[...] = jnp.dot(x, wnk_ref[...], preferred_element_type=jnp.float32)


def _escore_kernel(e_ref, we_ref, es_ref):
    es_ref[...] = jnp.dot(e_ref[...], we_ref[...],
                          preferred_element_type=jnp.float32)


def _thresh_kernel(ek_ref, nk_ref, ke_ref, kn_ref):
    def search(keys, kk):
        def body(_, carry):
            lo, hi = carry
            mid = (lo >> 1) + (hi >> 1) + (lo & hi & 1)
            cnt = jnp.sum((keys >= mid).astype(jnp.int32))
            ok = cnt >= kk
            return (jnp.where(ok, mid, lo), jnp.where(ok, hi, mid))
        lo, _ = jax.lax.fori_loop(
            0, 34, body, (jnp.full((), _INT_MIN), jnp.full((), _INT_MAX)))
        return lo
    ke_ref[...] = jnp.broadcast_to(search(ek_ref[...], _KE), (1, 1))
    kn_ref[...] = jnp.broadcast_to(search(nk_ref[...], _KN), (1, 1))


def _qkv_kernel(e_ref, mask_ref, xnd_ref, wq_ref, wk_ref, wv_ref, s_ref,
                we_ref, sc_ref, v_ref):
    we = e_ref[...] * mask_ref[...]
    q = jnp.dot(we, wq_ref[...], preferred_element_type=jnp.float32)
    k = jnp.dot(we, wk_ref[...], preferred_element_type=jnp.float32) \
        + xnd_ref[...]
    v = jnp.dot(we, wv_ref[...], preferred_element_type=jnp.float32)
    sc_ref[...] = jnp.dot(q * k, s_ref[...],
                          preferred_element_type=jnp.float32) * 0.5
    we_ref[...] = we
    v_ref[...] = v


def _exp_kernel(sc_ref, md_ref, v_ref, r_ref, ex_ref, exv_ref):
    ex = jnp.exp(sc_ref[...] - md_ref[...])
    ex_ref[...] = ex
    exr = jnp.dot(ex, r_ref[...], preferred_element_type=jnp.float32)
    exv_ref[...] = exr * v_ref[...]


def _out_kernel(we_ref, agg_ref, wo_ref, wc1_ref, bc1_ref, wc2_ref, bc2_ref,
                o_ref):
    upd = we_ref[...] + jnp.dot(agg_ref[...], wo_ref[...],
                                preferred_element_type=jnp.float32)
    h = jax.nn.gelu(jnp.dot(upd, wc1_ref[...],
                            preferred_element_type=jnp.float32) + bc1_ref[...])
    o_ref[...] = jnp.dot(h, wc2_ref[...],
                         preferred_element_type=jnp.float32) + bc2_ref[...]


def _key(f):
    i = jax.lax.bitcast_convert_type(f, jnp.int32)
    return jnp.where(i >= 0, i, _INT_MIN - i)


def kernel(node_features, edge_features, edge_index, node_tiers,
           wn_score, we_score, Wq, Wk, Wv, Wo, Wnk, Wc1, bc1, Wc2, bc2):
    x = node_features
    e = edge_features
    src = edge_index[0]
    dst = edge_index[1]

    ns, xn = pl.pallas_call(
        _node_kernel,
        grid=(_GN,),
        in_specs=[pl.BlockSpec((_BN, _DN), lambda i: (i, 0)),
                  pl.BlockSpec((_DN, 1), lambda i: (0, 0)),
                  pl.BlockSpec((_DN, _DE), lambda i: (0, 0))],
        out_specs=[pl.BlockSpec((_BN, 1), lambda i: (i, 0)),
                   pl.BlockSpec((_BN, _DE), lambda i: (i, 0))],
        out_shape=[jax.ShapeDtypeStruct((_N, 1), jnp.float32),
                   jax.ShapeDtypeStruct((_N, _DE), jnp.float32)],
    )(x, wn_score.reshape(_DN, 1), Wnk)

    es = pl.pallas_call(
        _escore_kernel,
        grid=(_GE,),
        in_specs=[pl.BlockSpec((_BE, _DE), lambda i: (i, 0)),
                  pl.BlockSpec((_DE, 1), lambda i: (0, 0))],
        out_specs=pl.BlockSpec((_BE, 1), lambda i: (i, 0)),
        out_shape=jax.ShapeDtypeStruct((_E, 1), jnp.float32),
    )(e, we_score.reshape(_DE, 1))

    ekeys = _key(es.reshape(2500, 128))
    ns_pad = jnp.concatenate(
        [ns.reshape(_N), jnp.full((_NPAD,), -jnp.inf, jnp.float32)]
    ).reshape(80, 128)
    nkeys = _key(ns_pad)

    ke, kn = pl.pallas_call(
        _thresh_kernel,
        out_shape=[jax.ShapeDtypeStruct((1, 1), jnp.int32),
                   jax.ShapeDtypeStruct((1, 1), jnp.int32)],
    )(ekeys, nkeys)

    node_mask = nkeys.reshape(-1)[:_N] >= kn[0, 0]
    emask = (ekeys.reshape(_E) >= ke[0, 0]) & node_mask[src] & node_mask[dst]
    maskf = emask.astype(jnp.float32)[:, None]
    xnd = xn[dst]

    we_arr, sc, v = pl.pallas_call(
        _qkv_kernel,
        grid=(_GE,),
        in_specs=[pl.BlockSpec((_BE, _DE), lambda i: (i, 0)),
                  pl.BlockSpec((_BE, 1), lambda i: (i, 0)),
                  pl.BlockSpec((_BE, _DE), lambda i: (i, 0)),
                  pl.BlockSpec((_DE, _DE), lambda i: (0, 0)),
                  pl.BlockSpec((_DE, _DE), lambda i: (0, 0)),
                  pl.BlockSpec((_DE, _DE), lambda i: (0, 0)),
                  pl.BlockSpec((_DE, _H), lambda i: (0, 0))],
        out_specs=[pl.BlockSpec((_BE, _DE), lambda i: (i, 0)),
                   pl.BlockSpec((_BE, _H), lambda i: (i, 0)),
                   pl.BlockSpec((_BE, _DE), lambda i: (i, 0))],
        out_shape=[jax.ShapeDtypeStruct((_E, _DE), jnp.float32),
                   jax.ShapeDtypeStruct((_E, _H), jnp.float32),
                   jax.ShapeDtypeStruct((_E, _DE), jnp.float32)],
    )(e, maskf, xnd, Wq, Wk, Wv, jnp.asarray(_S_np))

    m = jax.ops.segment_max(sc, dst, num_segments=_N)
    m = jnp.where(jnp.isfinite(m), m, 0.0)
    md = m[dst]

    ex, exv = pl.pallas_call(
        _exp_kernel,
        grid=(_GE,),
        in_specs=[pl.BlockSpec((_BE, _H), lambda i: (i, 0)),
                  pl.BlockSpec((_BE, _H), lambda i: (i, 0)),
                  pl.BlockSpec((_BE, _DE), lambda i: (i, 0)),
                  pl.BlockSpec((_H, _DE), lambda i: (0, 0))],
        out_specs=[pl.BlockSpec((_BE, _H), lambda i: (i, 0)),
                   pl.BlockSpec((_BE, _DE), lambda i: (i, 0))],
        out_shape=[jax.ShapeDtypeStruct((_E, _H), jnp.float32),
                   jax.ShapeDtypeStruct((_E, _DE), jnp.float32)],
    )(sc, md, v, jnp.asarray(_R_np))

    den = jax.ops.segment_sum(ex, dst, num_segments=_N)
    aggn = jax.ops.segment_sum(exv, dst, num_segments=_N)
    agg = aggn / (jnp.repeat(den, _DH, axis=1) + 1e-9)
    aggd = agg[dst]

    out = pl.pallas_call(
        _out_kernel,
        grid=(_GE,),
        in_specs=[pl.BlockSpec((_BE, _DE), lambda i: (i, 0)),
                  pl.BlockSpec((_BE, _DE), lambda i: (i, 0)),
                  pl.BlockSpec((_DE, _DE), lambda i: (0, 0)),
                  pl.BlockSpec((_DE, _DE), lambda i: (0, 0)),
                  pl.BlockSpec((1, _DE), lambda i: (0, 0)),
                  pl.BlockSpec((_DE, _C), lambda i: (0, 0)),
                  pl.BlockSpec((1, _C), lambda i: (0, 0))],
        out_specs=pl.BlockSpec((_BE, _C), lambda i: (i, 0)),
        out_shape=jax.ShapeDtypeStruct((_E, _C), jnp.float32),
    )(we_arr, aggd, Wo, Wc1, bc1.reshape(1, _DE), Wc2, bc2.reshape(1, _C))
    return out
